# Initial kernel scaffold; baseline (speedup 1.0000x reference)
import jax
import jax.numpy as jnp
from jax.experimental import pallas as pl


def kernel(g, h, W, b):
    raise NotImplementedError("write your pallas kernel here")


# speedup vs baseline: 2.6666x; 2.6666x over previous
import functools

import jax
import jax.numpy as jnp
from jax import lax
from jax.experimental import pallas as pl
from jax.experimental.pallas import tpu as pltpu


def _pool_body(g_ref, h_ref, w_ref, b_ref, gnew_ref, newh_ref, idx_ref, *, n, k):
    f32 = jnp.float32
    g2 = g_ref[0]
    h2 = h_ref[0]
    wv = w_ref[...]
    bv = b_ref[0, 0]

    w128 = jnp.broadcast_to(wv, (128, wv.shape[1]))
    s128 = jax.nn.sigmoid(
        lax.dot_general(h2, w128, (((1,), (1,)), ((), ())),
                        preferred_element_type=f32) + bv)
    s_col = s128[:, 0:1]
    s_row = jnp.transpose(s128)[0:1, :]

    i_col = lax.broadcasted_iota(jnp.int32, (n, 1), 0).astype(f32)
    i_row = lax.broadcasted_iota(jnp.int32, (1, n), 1).astype(f32)
    cmp = (s_row > s_col).astype(f32) + \
        ((s_row == s_col) & (i_row < i_col)).astype(f32)
    rank_col = jnp.sum(cmp, axis=1, keepdims=True)
    cmp2 = (s_col > s_row).astype(f32) + \
        ((s_col == s_row) & (i_col < i_row)).astype(f32)
    rank_row = jnp.sum(cmp2, axis=0, keepdims=True)

    k_row = lax.broadcasted_iota(jnp.int32, (1, k), 1).astype(f32)
    k_col = lax.broadcasted_iota(jnp.int32, (k, 1), 0).astype(f32)
    pt = (rank_col == k_row).astype(f32)
    p = (rank_row == k_col).astype(f32)

    idx_row = jnp.sum(pt * i_col, axis=0, keepdims=True)
    vals = jnp.sum(p * s_row, axis=1, keepdims=True)

    dstd = (((1,), (0,)), ((), ()))
    hsel = lax.dot_general(p, h2, dstd, preferred_element_type=f32)
    newh_ref[0] = hsel * vals

    bf16 = jnp.bfloat16
    gb = g2.astype(bf16)
    gr = lax.dot_general(p.astype(bf16), gb, dstd,
                         preferred_element_type=f32)
    gc = lax.dot_general(gb, pt.astype(bf16), dstd,
                         preferred_element_type=f32)
    m = lax.dot_general(gr.astype(bf16), gc.astype(bf16), dstd,
                        preferred_element_type=f32)
    th = (m > 0.5).astype(f32)
    deg = jnp.sum(th, axis=1, keepdims=True)
    gnew_ref[0] = th / deg
    idx_ref[0] = idx_row.astype(jnp.int32)


def kernel(g, h, W, b):
    B, N, _ = g.shape
    D = h.shape[-1]
    K = max(2, int(0.5 * N))
    b2 = b.reshape(1, 1).astype(jnp.float32)

    out = pl.pallas_call(
        functools.partial(_pool_body, n=N, k=K),
        grid=(B,),
        in_specs=[
            pl.BlockSpec((1, N, N), lambda i: (i, 0, 0)),
            pl.BlockSpec((1, N, D), lambda i: (i, 0, 0)),
            pl.BlockSpec((1, D), lambda i: (0, 0)),
            pl.BlockSpec((1, 1), lambda i: (0, 0)),
        ],
        out_specs=[
            pl.BlockSpec((1, K, K), lambda i: (i, 0, 0)),
            pl.BlockSpec((1, K, D), lambda i: (i, 0, 0)),
            pl.BlockSpec((1, 1, K), lambda i: (i, 0, 0)),
        ],
        out_shape=[
            jax.ShapeDtypeStruct((B, K, K), jnp.float32),
            jax.ShapeDtypeStruct((B, K, D), jnp.float32),
            jax.ShapeDtypeStruct((B, 1, K), jnp.int32),
        ],
    )(g, h, W, b2)
    g_new, new_h, idx = out
    return (g_new, new_h, idx.reshape(B, K))


# speedup vs baseline: 3.2518x; 1.2195x over previous
import functools

import jax
import jax.numpy as jnp
from jax import lax
from jax.experimental import pallas as pl
from jax.experimental.pallas import tpu as pltpu


def _pool_body(g_ref, h_ref, w_ref, b_ref, gnew_ref, newh_ref, idx_ref, *, n, k):
    f32 = jnp.float32
    g2 = g_ref[0]
    h2 = h_ref[0]
    wv = w_ref[...]
    bv = b_ref[0, 0]

    w128 = jnp.broadcast_to(wv, (128, wv.shape[1]))
    s128 = jax.nn.sigmoid(
        lax.dot_general(h2, w128, (((1,), (1,)), ((), ())),
                        preferred_element_type=f32) + bv)
    s_col = s128[:, 0:1]
    s_row = jnp.transpose(s128)[0:1, :]

    bf16 = jnp.bfloat16
    i_col = lax.broadcasted_iota(jnp.int32, (n, 1), 0).astype(f32)
    i_row = lax.broadcasted_iota(jnp.int32, (1, n), 1).astype(f32)
    beats = (s_row > s_col) | ((s_row == s_col) & (i_row < i_col))
    cmpf = beats.astype(f32)
    ones128 = jnp.ones((n, 128), f32)
    rank128 = lax.dot_general(cmpf, ones128, (((1,), (0,)), ((), ())),
                              preferred_element_type=f32)
    rank_col = rank128[:, 0:1]
    rank_row = jnp.transpose(rank128)[0:1, :]

    k_row = lax.broadcasted_iota(jnp.int32, (1, k), 1).astype(f32)
    k_col = lax.broadcasted_iota(jnp.int32, (k, 1), 0).astype(f32)
    pt = (rank_col == k_row).astype(f32)
    p = (rank_row == k_col).astype(f32)

    idx_row = jnp.sum(pt * i_col, axis=0, keepdims=True)
    vals = jnp.sum(p * s_row, axis=1, keepdims=True)

    dstd = (((1,), (0,)), ((), ()))
    hsel = lax.dot_general(p, h2, dstd, preferred_element_type=f32)
    newh_ref[0] = hsel * vals

    gb = g2.astype(bf16)
    gr = lax.dot_general(p.astype(bf16), gb, dstd,
                         preferred_element_type=f32)
    gc = lax.dot_general(gb, pt.astype(bf16), dstd,
                         preferred_element_type=f32)
    m = lax.dot_general(gr.astype(bf16), gc.astype(bf16), dstd,
                        preferred_element_type=f32)
    th = (m > 0.5).astype(f32)
    deg = jnp.sum(th, axis=1, keepdims=True)
    gnew_ref[0] = th / deg
    idx_ref[0] = idx_row.astype(jnp.int32)


def kernel(g, h, W, b):
    B, N, _ = g.shape
    D = h.shape[-1]
    K = max(2, int(0.5 * N))
    b2 = b.reshape(1, 1).astype(jnp.float32)

    out = pl.pallas_call(
        functools.partial(_pool_body, n=N, k=K),
        grid=(B,),
        in_specs=[
            pl.BlockSpec((1, N, N), lambda i: (i, 0, 0)),
            pl.BlockSpec((1, N, D), lambda i: (i, 0, 0)),
            pl.BlockSpec((1, D), lambda i: (0, 0)),
            pl.BlockSpec((1, 1), lambda i: (0, 0)),
        ],
        out_specs=[
            pl.BlockSpec((1, K, K), lambda i: (i, 0, 0)),
            pl.BlockSpec((1, K, D), lambda i: (i, 0, 0)),
            pl.BlockSpec((1, 1, K), lambda i: (i, 0, 0)),
        ],
        out_shape=[
            jax.ShapeDtypeStruct((B, K, K), jnp.float32),
            jax.ShapeDtypeStruct((B, K, D), jnp.float32),
            jax.ShapeDtypeStruct((B, 1, K), jnp.int32),
        ],
    )(g, h, W, b2)
    g_new, new_h, idx = out
    return (g_new, new_h, idx.reshape(B, K))
